# baseline (speedup 1.0000x reference)
import functools

import jax
import jax.numpy as jnp
from jax import lax
from jax.experimental import pallas as pl
from jax.experimental.pallas import tpu as pltpu
from jax.experimental.pallas import tpu_sc as plsc

F = 128
D_EXT = 144
NC, NS, L = 2, 16, 16
NW = NC * NS
B_E = 40
RB = 1024

_SC_PARAMS = pltpu.CompilerParams(
    needs_layout_passes=False, use_tc_tiling_on_sc=False)


def _proj_body(x_ref, w_ref, al_ref, ar_ref, fe_ref, el_ref, er_ref):
    f = jnp.dot(x_ref[...], w_ref[...], preferred_element_type=jnp.float32)
    ind = jnp.where(
        lax.broadcasted_iota(jnp.int32, (RB, D_EXT - F), 1) == 0, 1.0, 0.0
    ).astype(jnp.float32)
    fe_ref[...] = jnp.concatenate([f, ind], axis=1)
    el_ref[...] = jnp.sum(f * al_ref[...], axis=1)
    er_ref[...] = jnp.sum(f * ar_ref[...], axis=1)


def _comb_body(p_ref, b_ref, o_ref):
    m = p_ref[0] + p_ref[1]
    msg = m[:, 0:F]
    den = m[:, F:F + 1]
    o_ref[...] = msg / (den + jnp.float32(1e-9)) + b_ref[...]


def _mesh():
    return plsc.VectorSubcoreMesh(
        core_axis_name="c", subcore_axis_name="s",
        num_cores=NC, num_subcores=NS)


def _make_sc_coeff(n_pad, ept, e_tot):
    @functools.partial(
        pl.kernel,
        out_type=jax.ShapeDtypeStruct((e_tot,), jnp.float32),
        mesh=_mesh(),
        compiler_params=_SC_PARAMS,
        scratch_types=[
            pltpu.VMEM((n_pad,), jnp.float32),
            pltpu.VMEM((n_pad,), jnp.float32),
            pltpu.VMEM((ept,), jnp.int32),
            pltpu.VMEM((ept,), jnp.int32),
            pltpu.VMEM((ept,), jnp.float32),
        ],
    )
    def sc_coeff(elp_hbm, erp_hbm, src_hbm, dst_hbm, ee_hbm,
                 el_v, er_v, src_f, dst_f, ee_c):
        c = lax.axis_index("c")
        s = lax.axis_index("s")
        wid = c * NS + s
        ebase = wid * ept

        pltpu.sync_copy(elp_hbm, el_v)
        pltpu.sync_copy(erp_hbm, er_v)
        pltpu.sync_copy(src_hbm.at[pl.ds(ebase, ept)], src_f)
        pltpu.sync_copy(dst_hbm.at[pl.ds(ebase, ept)], dst_f)

        def eebody(g, _):
            for u in range(4):
                b = g * (4 * L) + u * L
                si = src_f[pl.ds(b, L)]
                di = dst_f[pl.ds(b, L)]
                e = plsc.load_gather(el_v, [si]) + plsc.load_gather(er_v, [di])
                e = jnp.where(e >= 0.0, e, e * jnp.float32(0.2))
                ee_c[pl.ds(b, L)] = jnp.exp(e)
            return 0
        lax.fori_loop(0, ept // (4 * L), eebody, 0)

        pltpu.sync_copy(ee_c, ee_hbm.at[pl.ds(ebase, ept)])

    return sc_coeff


def _make_sc_scatter(n, ept, nbt):
    chunk = n // NS
    K = 5
    assert nbt % K == 0

    @functools.partial(
        pl.kernel,
        out_type=jax.ShapeDtypeStruct((NC, n, D_EXT), jnp.float32),
        mesh=_mesh(),
        compiler_params=_SC_PARAMS,
        scratch_types=(
            [pltpu.VMEM((ept,), jnp.int32)]
            + [pltpu.VMEM((B_E, D_EXT), jnp.float32)] * K
            + [pltpu.VMEM((B_E,), jnp.float32)] * K
            + [pltpu.VMEM((B_E,), jnp.int32)] * K
            + [pltpu.VMEM_SHARED((n, D_EXT), jnp.float32)]
            + [pltpu.SemaphoreType.DMA] * (2 * K)
        ),
    )
    def sc_scatter(fe_hbm, src_hbm, dst_hbm, ee_hbm, z_hbm, out_hbm,
                   src_f, *rest):
        bufs = rest[0:K]
        eebs = rest[K:2 * K]
        dstbs = rest[2 * K:3 * K]
        out_sp = rest[3 * K]
        sem_g = rest[3 * K + 1:3 * K + 1 + K]
        sem_s = rest[3 * K + 1 + K:3 * K + 1 + 2 * K]

        c = lax.axis_index("c")
        s = lax.axis_index("s")
        wid = c * NS + s
        ebase = wid * ept

        pltpu.sync_copy(src_hbm.at[pl.ds(ebase, ept)], src_f)
        pltpu.sync_copy(z_hbm.at[pl.ds(s * chunk, chunk)],
                        out_sp.at[pl.ds(s * chunk, chunk)])
        plsc.subcore_barrier()

        def start_gather(j, t):
            pltpu.async_copy(
                fe_hbm.at[src_f.at[pl.ds(j * B_E, B_E)]], bufs[t], sem_g[t])
            pltpu.async_copy(
                ee_hbm.at[pl.ds(ebase + j * B_E, B_E)], eebs[t], sem_g[t])
            pltpu.async_copy(
                dst_hbm.at[pl.ds(ebase + j * B_E, B_E)], dstbs[t], sem_g[t])

        def wait_gather(j, t):
            pltpu.make_async_copy(
                fe_hbm.at[src_f.at[pl.ds(j * B_E, B_E)]], bufs[t],
                sem_g[t]).wait()
            pltpu.make_async_copy(
                ee_hbm.at[pl.ds(ebase + j * B_E, B_E)], eebs[t],
                sem_g[t]).wait()
            pltpu.make_async_copy(
                dst_hbm.at[pl.ds(ebase + j * B_E, B_E)], dstbs[t],
                sem_g[t]).wait()

        def start_scatter(t):
            pltpu.async_copy(bufs[t], out_sp.at[dstbs[t]], sem_s[t],
                             add=True)

        def wait_scatter(t):
            pltpu.make_async_copy(bufs[t], out_sp.at[dstbs[t]],
                                  sem_s[t]).wait()

        jzero = (c * 0).astype(jnp.int32)

        def scale(buf, eeb):
            zr = jnp.full((L,), jzero, jnp.int32)
            den_msk = (lax.broadcasted_iota(jnp.int32, (L,), 0) == 0).astype(
                jnp.float32)

            def edge(e_t, ee16, u):
                bc = ee16.at[zr + u].get(mode="promise_in_bounds")
                for m in range(F // L):
                    sl = pl.ds(m * L, L)
                    buf[e_t, sl] = buf[e_t, sl] * bc
                buf[e_t, pl.ds(F, L)] = bc * den_msk

            def hbody(h, _):
                base = h * L
                ee16 = eeb[pl.ds(base, L)]
                for u in range(L):
                    edge(base + u, ee16, u)
                return 0
            lax.fori_loop(0, B_E // L, hbody, 0)
            ee16 = eeb[pl.ds(B_E - L, L)]
            for u in range(L - (B_E - (B_E // L) * L), L):
                edge(B_E - L + u, ee16, u)

        for t in range(K - 1):
            start_gather(t, t)

        def mbody(i, _):
            for t in range(K):
                j = i * K + t
                wait_gather(j, t)
                scale(bufs[t], eebs[t])
                start_scatter(t)
                tn = (t + K - 1) % K

                @pl.when(j >= 1)
                def _():
                    wait_scatter(tn)

                @pl.when(j + K - 1 < nbt)
                def _():
                    start_gather(j + K - 1, tn)
            return 0
        lax.fori_loop(0, nbt // K, mbody, 0)
        wait_scatter((nbt - 1) % K)

        plsc.subcore_barrier()
        pltpu.sync_copy(out_sp.at[pl.ds(s * chunk, chunk)],
                        out_hbm.at[c, pl.ds(s * chunk, chunk)])

    return sc_scatter


def kernel(x, edge_index, W, attn_l, attn_r, bias):
    n, _ = x.shape
    e_tot = edge_index.shape[1]
    n_pad = ((n + NS * L - 1) // (NS * L)) * (NS * L)
    ept = e_tot // NW
    nbt = ept // B_E

    feat_ext, el2, er2 = pl.pallas_call(
        _proj_body,
        grid=((n + RB - 1) // RB,),
        in_specs=[
            pl.BlockSpec((RB, F), lambda i: (i, 0)),
            pl.BlockSpec((F, F), lambda i: (0, 0)),
            pl.BlockSpec((1, F), lambda i: (0, 0)),
            pl.BlockSpec((1, F), lambda i: (0, 0)),
        ],
        out_specs=[
            pl.BlockSpec((RB, D_EXT), lambda i: (i, 0)),
            pl.BlockSpec((RB,), lambda i: (i,)),
            pl.BlockSpec((RB,), lambda i: (i,)),
        ],
        out_shape=[
            jax.ShapeDtypeStruct((n, D_EXT), jnp.float32),
            jax.ShapeDtypeStruct((n_pad,), jnp.float32),
            jax.ShapeDtypeStruct((n_pad,), jnp.float32),
        ],
    )(x, W, attn_l.reshape(1, F), attn_r.reshape(1, F))

    src = edge_index[0]
    dst = edge_index[1]
    z = jnp.zeros((n, D_EXT), jnp.float32)

    ee = _make_sc_coeff(n_pad, ept, e_tot)(el2, er2, src, dst)
    outp = _make_sc_scatter(n, ept, nbt)(feat_ext, src, dst, ee, z)

    out = pl.pallas_call(
        _comb_body,
        grid=((n + RB - 1) // RB,),
        in_specs=[
            pl.BlockSpec((NC, RB, D_EXT), lambda i: (0, i, 0)),
            pl.BlockSpec((1, F), lambda i: (0, 0)),
        ],
        out_specs=pl.BlockSpec((RB, F), lambda i: (i, 0)),
        out_shape=jax.ShapeDtypeStruct((n, F), jnp.float32),
    )(outp, bias.reshape(1, F))
    return out


# speedup vs baseline: 1.0049x; 1.0049x over previous
import functools

import jax
import jax.numpy as jnp
from jax import lax
from jax.experimental import pallas as pl
from jax.experimental.pallas import tpu as pltpu
from jax.experimental.pallas import tpu_sc as plsc

F = 128
D_EXT = 144
NC, NS, L = 2, 16, 16
NW = NC * NS
B_E = 40
RB = 1024

_SC_PARAMS = pltpu.CompilerParams(
    needs_layout_passes=False, use_tc_tiling_on_sc=False)


def _proj_body(x_ref, w_ref, al_ref, ar_ref, fe_ref, el_ref, er_ref):
    f = jnp.dot(x_ref[...], w_ref[...], preferred_element_type=jnp.float32)
    ind = jnp.where(
        lax.broadcasted_iota(jnp.int32, (RB, D_EXT - F), 1) == 0, 1.0, 0.0
    ).astype(jnp.float32)
    fe_ref[...] = jnp.concatenate([f, ind], axis=1)
    el_ref[...] = jnp.sum(f * al_ref[...], axis=1)
    er_ref[...] = jnp.sum(f * ar_ref[...], axis=1)


def _comb_body(p_ref, b_ref, o_ref):
    m = p_ref[0] + p_ref[1]
    msg = m[:, 0:F]
    den = m[:, F:F + 1]
    o_ref[...] = msg / (den + jnp.float32(1e-9)) + b_ref[...]


def _mesh():
    return plsc.VectorSubcoreMesh(
        core_axis_name="c", subcore_axis_name="s",
        num_cores=NC, num_subcores=NS)


def _make_sc_coeff(n_pad, ept, e_tot):
    @functools.partial(
        pl.kernel,
        out_type=jax.ShapeDtypeStruct((e_tot,), jnp.float32),
        mesh=_mesh(),
        compiler_params=_SC_PARAMS,
        scratch_types=[
            pltpu.VMEM((n_pad,), jnp.float32),
            pltpu.VMEM((n_pad,), jnp.float32),
            pltpu.VMEM((ept,), jnp.int32),
            pltpu.VMEM((ept,), jnp.int32),
            pltpu.VMEM((ept,), jnp.float32),
        ],
    )
    def sc_coeff(elp_hbm, erp_hbm, src_hbm, dst_hbm, ee_hbm,
                 el_v, er_v, src_f, dst_f, ee_c):
        c = lax.axis_index("c")
        s = lax.axis_index("s")
        wid = c * NS + s
        ebase = wid * ept

        pltpu.sync_copy(elp_hbm, el_v)
        pltpu.sync_copy(erp_hbm, er_v)
        pltpu.sync_copy(src_hbm.at[pl.ds(ebase, ept)], src_f)
        pltpu.sync_copy(dst_hbm.at[pl.ds(ebase, ept)], dst_f)

        def eebody(g, _):
            b = g * L
            si = src_f[pl.ds(b, L)]
            di = dst_f[pl.ds(b, L)]
            e = plsc.load_gather(el_v, [si]) + plsc.load_gather(er_v, [di])
            e = jnp.where(e >= 0.0, e, e * jnp.float32(0.2))
            ee_c[pl.ds(b, L)] = jnp.exp(e)
            return 0
        lax.fori_loop(0, ept // L, eebody, 0)

        pltpu.sync_copy(ee_c, ee_hbm.at[pl.ds(ebase, ept)])

    return sc_coeff


def _make_sc_scatter(n, ept, nbt):
    chunk = n // NS
    K = 5
    assert nbt % K == 0

    @functools.partial(
        pl.kernel,
        out_type=jax.ShapeDtypeStruct((NC, n, D_EXT), jnp.float32),
        mesh=_mesh(),
        compiler_params=_SC_PARAMS,
        scratch_types=(
            [pltpu.VMEM((ept,), jnp.int32)]
            + [pltpu.VMEM((B_E, D_EXT), jnp.float32)] * K
            + [pltpu.VMEM((B_E,), jnp.float32)] * K
            + [pltpu.VMEM((B_E,), jnp.int32)] * K
            + [pltpu.VMEM_SHARED((n, D_EXT), jnp.float32)]
            + [pltpu.SemaphoreType.DMA] * (2 * K)
        ),
    )
    def sc_scatter(fe_hbm, src_hbm, dst_hbm, ee_hbm, z_hbm, out_hbm,
                   src_f, *rest):
        bufs = rest[0:K]
        eebs = rest[K:2 * K]
        dstbs = rest[2 * K:3 * K]
        out_sp = rest[3 * K]
        sem_g = rest[3 * K + 1:3 * K + 1 + K]
        sem_s = rest[3 * K + 1 + K:3 * K + 1 + 2 * K]

        c = lax.axis_index("c")
        s = lax.axis_index("s")
        wid = c * NS + s
        ebase = wid * ept

        pltpu.sync_copy(src_hbm.at[pl.ds(ebase, ept)], src_f)
        pltpu.sync_copy(z_hbm.at[pl.ds(s * chunk, chunk)],
                        out_sp.at[pl.ds(s * chunk, chunk)])
        plsc.subcore_barrier()

        def start_gather(j, t):
            pltpu.async_copy(
                fe_hbm.at[src_f.at[pl.ds(j * B_E, B_E)]], bufs[t], sem_g[t])
            pltpu.async_copy(
                ee_hbm.at[pl.ds(ebase + j * B_E, B_E)], eebs[t], sem_g[t])
            pltpu.async_copy(
                dst_hbm.at[pl.ds(ebase + j * B_E, B_E)], dstbs[t], sem_g[t])

        def wait_gather(j, t):
            pltpu.make_async_copy(
                fe_hbm.at[src_f.at[pl.ds(j * B_E, B_E)]], bufs[t],
                sem_g[t]).wait()
            pltpu.make_async_copy(
                ee_hbm.at[pl.ds(ebase + j * B_E, B_E)], eebs[t],
                sem_g[t]).wait()
            pltpu.make_async_copy(
                dst_hbm.at[pl.ds(ebase + j * B_E, B_E)], dstbs[t],
                sem_g[t]).wait()

        def start_scatter(t):
            pltpu.async_copy(bufs[t], out_sp.at[dstbs[t]], sem_s[t],
                             add=True)

        def wait_scatter(t):
            pltpu.make_async_copy(bufs[t], out_sp.at[dstbs[t]],
                                  sem_s[t]).wait()

        jzero = (c * 0).astype(jnp.int32)

        def scale(buf, eeb):
            zr = jnp.full((L,), jzero, jnp.int32)
            den_msk = (lax.broadcasted_iota(jnp.int32, (L,), 0) == 0).astype(
                jnp.float32)

            def edge(e_t, ee16, u):
                bc = ee16.at[zr + u].get(mode="promise_in_bounds")
                for m in range(F // L):
                    sl = pl.ds(m * L, L)
                    buf[e_t, sl] = buf[e_t, sl] * bc
                buf[e_t, pl.ds(F, L)] = bc * den_msk

            def hbody(h, _):
                base = h * L
                ee16 = eeb[pl.ds(base, L)]
                for u in range(L):
                    edge(base + u, ee16, u)
                return 0
            lax.fori_loop(0, B_E // L, hbody, 0)
            ee16 = eeb[pl.ds(B_E - L, L)]
            for u in range(L - (B_E - (B_E // L) * L), L):
                edge(B_E - L + u, ee16, u)

        for t in range(K - 1):
            start_gather(t, t)

        def mbody(i, _):
            for t in range(K):
                j = i * K + t
                wait_gather(j, t)
                scale(bufs[t], eebs[t])
                start_scatter(t)
                tn = (t + K - 1) % K

                @pl.when(j >= 1)
                def _():
                    wait_scatter(tn)

                @pl.when(j + K - 1 < nbt)
                def _():
                    start_gather(j + K - 1, tn)
            return 0
        lax.fori_loop(0, nbt // K, mbody, 0)
        wait_scatter((nbt - 1) % K)

        plsc.subcore_barrier()
        pltpu.sync_copy(out_sp.at[pl.ds(s * chunk, chunk)],
                        out_hbm.at[c, pl.ds(s * chunk, chunk)])

    return sc_scatter


def kernel(x, edge_index, W, attn_l, attn_r, bias):
    n, _ = x.shape
    e_tot = edge_index.shape[1]
    n_pad = ((n + NS * L - 1) // (NS * L)) * (NS * L)
    ept = e_tot // NW
    nbt = ept // B_E

    feat_ext, el2, er2 = pl.pallas_call(
        _proj_body,
        grid=((n + RB - 1) // RB,),
        in_specs=[
            pl.BlockSpec((RB, F), lambda i: (i, 0)),
            pl.BlockSpec((F, F), lambda i: (0, 0)),
            pl.BlockSpec((1, F), lambda i: (0, 0)),
            pl.BlockSpec((1, F), lambda i: (0, 0)),
        ],
        out_specs=[
            pl.BlockSpec((RB, D_EXT), lambda i: (i, 0)),
            pl.BlockSpec((RB,), lambda i: (i,)),
            pl.BlockSpec((RB,), lambda i: (i,)),
        ],
        out_shape=[
            jax.ShapeDtypeStruct((n, D_EXT), jnp.float32),
            jax.ShapeDtypeStruct((n_pad,), jnp.float32),
            jax.ShapeDtypeStruct((n_pad,), jnp.float32),
        ],
    )(x, W, attn_l.reshape(1, F), attn_r.reshape(1, F))

    src = edge_index[0]
    dst = edge_index[1]
    z = jnp.zeros((n, D_EXT), jnp.float32)

    ee = _make_sc_coeff(n_pad, ept, e_tot)(el2, er2, src, dst)
    outp = _make_sc_scatter(n, ept, nbt)(feat_ext, src, dst, ee, z)

    out = pl.pallas_call(
        _comb_body,
        grid=((n + RB - 1) // RB,),
        in_specs=[
            pl.BlockSpec((NC, RB, D_EXT), lambda i: (0, i, 0)),
            pl.BlockSpec((1, F), lambda i: (0, 0)),
        ],
        out_specs=pl.BlockSpec((RB, F), lambda i: (i, 0)),
        out_shape=jax.ShapeDtypeStruct((n, F), jnp.float32),
    )(outp, bias.reshape(1, F))
    return out
